# baseline (speedup 1.0000x reference)
import functools

import jax
import jax.numpy as jnp
from jax import lax
from jax.experimental import pallas as pl
from jax.experimental.pallas import tpu as pltpu
from jax.experimental.pallas import tpu_sc as plsc

B, N, M, K = 16, 2048, 2048, 3
M_SC = 384
M_TC = M - M_SC
NC, NS = 2, 16
N_H = N // 2
NV = N_H // 16


def _tc_kernel(x1_ref, x2t_ref, rowmin_ref, s2_ref, s2acc_ref):
    b = pl.program_id(0)

    x1 = x1_ref[0]
    x2t = x2t_ref[0]

    a0 = x1[:, 0:1]
    a1 = x1[:, 1:2]
    a2 = x1[:, 2:3]
    b0 = x2t[0:1, :]
    b1 = x2t[1:2, :]
    b2 = x2t[2:3, :]

    d0 = a0 - b0
    dist = d0 * d0
    d1 = a1 - b1
    dist = d1 * d1 + dist
    d2 = a2 - b2
    dist = d2 * d2 + dist

    rowmin_ref[0, 0, :] = jnp.min(dist, axis=1)
    col_min = jnp.min(dist, axis=0, keepdims=True)

    @pl.when(b == 0)
    def _init():
        s2acc_ref[0] = 0.0

    s2acc_ref[0] += jnp.sum(col_min)

    @pl.when(b == B - 1)
    def _finish():
        s2_ref[0, 0] = s2acc_ref[0]


def _sc_kernel(x1t_hbm, x2t_hbm, rowmin_hbm, colmin_hbm,
               x1_v, x2_v, rm_v, cm_v):
    c = lax.axis_index("c")
    s = lax.axis_index("s")
    b = s
    h = c

    pltpu.sync_copy(x1t_hbm.at[b, :, pl.ds(h * N_H, N_H)], x1_v)
    pltpu.sync_copy(x2t_hbm.at[b, :, pl.ds(M_TC, M_SC)], x2_v)

    inf16 = jnp.full((16,), jnp.inf, jnp.float32)

    def init_rm(i, carry):
        rm_v[pl.ds(i * 16, 16)] = inf16
        return carry

    lax.fori_loop(0, NV, init_rm, 0)

    def mg_body(mg, carry):
        msl = pl.ds(mg * 16, 16)
        b0v = x2_v[0, msl]
        b1v = x2_v[1, msl]
        b2v = x2_v[2, msl]
        for jj in range(2):
            bs = [(b0v[jj * 8 + t], b1v[jj * 8 + t], b2v[jj * 8 + t])
                  for t in range(8)]

            @plsc.parallel_loop(0, NV, 1, unroll=4,
                                carry=(inf16,) * 8)
            def cmins(i, cmins):
                sl = pl.ds(i * 16, 16)
                a0 = x1_v[0, sl]
                a1 = x1_v[1, sl]
                a2 = x1_v[2, sl]
                ds = []
                for t in range(8):
                    b0, b1, b2 = bs[t]
                    d0 = a0 - b0
                    d1 = a1 - b1
                    d2 = a2 - b2
                    ds.append(d0 * d0 + d1 * d1 + d2 * d2)
                m03 = jnp.minimum(jnp.minimum(ds[0], ds[1]),
                                  jnp.minimum(ds[2], ds[3]))
                m47 = jnp.minimum(jnp.minimum(ds[4], ds[5]),
                                  jnp.minimum(ds[6], ds[7]))
                rm_v[sl] = jnp.minimum(rm_v[sl], jnp.minimum(m03, m47))
                return tuple(jnp.minimum(c, d) for c, d in zip(cmins, ds))
            for t in range(8):
                cm_v[pl.ds((mg * 16 + jj * 8 + t) * 16, 16)] = cmins[t]
        return carry

    lax.fori_loop(0, M_SC // 16, mg_body, 0)

    wid = s * NC + c
    pltpu.sync_copy(rm_v, rowmin_hbm.at[b, pl.ds(h * N_H, N_H)])
    pltpu.sync_copy(cm_v, colmin_hbm.at[wid])


_sc_chamfer = functools.partial(
    pl.kernel,
    out_type=[
        jax.ShapeDtypeStruct((B, N), jnp.float32),
        jax.ShapeDtypeStruct((NC * NS, M_SC * 16), jnp.float32),
    ],
    mesh=plsc.VectorSubcoreMesh(core_axis_name="c", subcore_axis_name="s"),
    scratch_types=[
        pltpu.VMEM((K, N_H), jnp.float32),
        pltpu.VMEM((K, M_SC), jnp.float32),
        pltpu.VMEM((N_H,), jnp.float32),
        pltpu.VMEM((M_SC * 16,), jnp.float32),
    ],
)(_sc_kernel)


@jax.jit
def kernel(xyz1, xyz2):
    x1t = jnp.transpose(xyz1, (0, 2, 1))
    x2t = jnp.transpose(xyz2, (0, 2, 1))

    rm_sc, cm_sc = _sc_chamfer(x1t, x2t)

    rowmin_tc, s2_tc = pl.pallas_call(
        _tc_kernel,
        grid=(B,),
        in_specs=[
            pl.BlockSpec((1, N, K), lambda b: (b, 0, 0)),
            pl.BlockSpec((1, K, M_TC), lambda b: (b, 0, 0)),
        ],
        out_specs=[
            pl.BlockSpec((1, 1, N), lambda b: (b, 0, 0)),
            pl.BlockSpec((1, 1), lambda b: (0, 0), memory_space=pltpu.SMEM),
        ],
        out_shape=[
            jax.ShapeDtypeStruct((B, 1, N), jnp.float32),
            jax.ShapeDtypeStruct((1, 1), jnp.float32),
        ],
        scratch_shapes=[pltpu.SMEM((1,), jnp.float32)],
    )(xyz1, x2t[:, :, :M_TC])

    row_min = jnp.minimum(rowmin_tc[:, 0, :], rm_sc)
    s1 = jnp.sum(row_min)
    s2 = s2_tc[0, 0] + jnp.sum(
        jnp.min(cm_sc.reshape(B, 2, M_SC, 16), axis=(1, 3)))
    return s1 / (B * N) + s2 / (B * M)


# speedup vs baseline: 1.0826x; 1.0826x over previous
import functools

import jax
import jax.numpy as jnp
from jax import lax
from jax.experimental import pallas as pl
from jax.experimental.pallas import tpu as pltpu
from jax.experimental.pallas import tpu_sc as plsc

B, N, M, K = 16, 2048, 2048, 3
M_SC = 512
M_TC = M - M_SC
NC, NS = 2, 16
M_W = M_SC // 2
NV = N // 16


def _tc_kernel(x1_ref, x2t_ref, rowmin_ref, s2_ref, s2acc_ref):
    b = pl.program_id(0)

    x1 = x1_ref[0]
    x2t = x2t_ref[0]

    a0 = x1[:, 0:1]
    a1 = x1[:, 1:2]
    a2 = x1[:, 2:3]
    b0 = x2t[0:1, :]
    b1 = x2t[1:2, :]
    b2 = x2t[2:3, :]

    d0 = a0 - b0
    dist = d0 * d0
    d1 = a1 - b1
    dist = d1 * d1 + dist
    d2 = a2 - b2
    dist = d2 * d2 + dist

    rowmin_ref[0, 0, :] = jnp.min(dist, axis=1)
    col_min = jnp.min(dist, axis=0, keepdims=True)

    @pl.when(b == 0)
    def _init():
        s2acc_ref[0] = 0.0

    s2acc_ref[0] += jnp.sum(col_min)

    @pl.when(b == B - 1)
    def _finish():
        s2_ref[0, 0] = s2acc_ref[0]


def _sc_kernel(x1t_hbm, x2t_hbm, rowmin_hbm, colmin_hbm,
               x1_v, x2_v, rm_v, cm_v):
    c = lax.axis_index("c")
    s = lax.axis_index("s")
    b = s
    h = c

    pltpu.sync_copy(x1t_hbm.at[b], x1_v)
    pltpu.sync_copy(x2t_hbm.at[b, :, pl.ds(M_TC + h * M_W, M_W)],
                    x2_v)

    inf16 = jnp.full((16,), jnp.inf, jnp.float32)

    def init_rm(i, carry):
        rm_v[pl.ds(i * 16, 16)] = inf16
        return carry

    lax.fori_loop(0, NV, init_rm, 0)

    def mg_body(mg, carry):
        msl = pl.ds(mg * 16, 16)
        b0v = x2_v[0, msl]
        b1v = x2_v[1, msl]
        b2v = x2_v[2, msl]
        for jj in range(2):
            bs = [(b0v[jj * 8 + t], b1v[jj * 8 + t], b2v[jj * 8 + t])
                  for t in range(8)]

            @plsc.parallel_loop(0, NV, 1, unroll=4,
                                carry=(inf16,) * 8)
            def cmins(i, cmins):
                sl = pl.ds(i * 16, 16)
                a0 = x1_v[0, sl]
                a1 = x1_v[1, sl]
                a2 = x1_v[2, sl]
                ds = []
                for t in range(8):
                    b0, b1, b2 = bs[t]
                    d0 = a0 - b0
                    d1 = a1 - b1
                    d2 = a2 - b2
                    ds.append(d0 * d0 + d1 * d1 + d2 * d2)
                m03 = jnp.minimum(jnp.minimum(ds[0], ds[1]),
                                  jnp.minimum(ds[2], ds[3]))
                m47 = jnp.minimum(jnp.minimum(ds[4], ds[5]),
                                  jnp.minimum(ds[6], ds[7]))
                rm_v[sl] = jnp.minimum(rm_v[sl], jnp.minimum(m03, m47))
                return tuple(jnp.minimum(c, d) for c, d in zip(cmins, ds))
            for t in range(8):
                cm_v[pl.ds((mg * 16 + jj * 8 + t) * 16, 16)] = cmins[t]
        return carry

    lax.fori_loop(0, M_W // 16, mg_body, 0)

    wid = s * NC + c
    pltpu.sync_copy(rm_v, rowmin_hbm.at[wid])
    pltpu.sync_copy(cm_v, colmin_hbm.at[wid])


_sc_chamfer = functools.partial(
    pl.kernel,
    out_type=[
        jax.ShapeDtypeStruct((NC * NS, N), jnp.float32),
        jax.ShapeDtypeStruct((NC * NS, M_W * 16), jnp.float32),
    ],
    mesh=plsc.VectorSubcoreMesh(core_axis_name="c", subcore_axis_name="s"),
    scratch_types=[
        pltpu.VMEM((K, N), jnp.float32),
        pltpu.VMEM((K, M_W), jnp.float32),
        pltpu.VMEM((N,), jnp.float32),
        pltpu.VMEM((M_W * 16,), jnp.float32),
    ],
)(_sc_kernel)


@jax.jit
def kernel(xyz1, xyz2):
    x1t = jnp.transpose(xyz1, (0, 2, 1))
    x2t = jnp.transpose(xyz2, (0, 2, 1))

    rm_sc, cm_sc = _sc_chamfer(x1t, x2t)

    rowmin_tc, s2_tc = pl.pallas_call(
        _tc_kernel,
        grid=(B,),
        in_specs=[
            pl.BlockSpec((1, N, K), lambda b: (b, 0, 0)),
            pl.BlockSpec((1, K, M_TC), lambda b: (b, 0, 0)),
        ],
        out_specs=[
            pl.BlockSpec((1, 1, N), lambda b: (b, 0, 0)),
            pl.BlockSpec((1, 1), lambda b: (0, 0), memory_space=pltpu.SMEM),
        ],
        out_shape=[
            jax.ShapeDtypeStruct((B, 1, N), jnp.float32),
            jax.ShapeDtypeStruct((1, 1), jnp.float32),
        ],
        scratch_shapes=[pltpu.SMEM((1,), jnp.float32)],
    )(xyz1, x2t[:, :, :M_TC])

    rm_sc2 = jnp.min(rm_sc.reshape(B, 2, N), axis=1)
    row_min = jnp.minimum(rowmin_tc[:, 0, :], rm_sc2)
    s1 = jnp.sum(row_min)
    s2 = s2_tc[0, 0] + jnp.sum(jnp.min(cm_sc.reshape(-1, M_W, 16), axis=2))
    return s1 / (B * N) + s2 / (B * M)
